# Initial kernel scaffold; baseline (speedup 1.0000x reference)
import jax
import jax.numpy as jnp
from jax.experimental import pallas as pl


def kernel(x, G_edge_index, L_edge_index, W1_G, b1_G, W2_G, b2_G, Wm_G, bm_G, gamma_G, beta_G, mean_G, var_G, W1_L, b1_L, W2_L, b2_L, Wm_L, bm_L, gamma_L, beta_L, mean_L, var_L):
    raise NotImplementedError("write your pallas kernel here")


# speedup vs baseline: 6.7841x; 6.7841x over previous
import functools

import jax
import jax.numpy as jnp
from jax import lax
from jax.experimental import pallas as pl
from jax.experimental.pallas import tpu as pltpu
from jax.experimental.pallas import tpu_sc as plsc

N = 10000
E = 320000
D = 128
H = 256
Z = 128
K = 3

NPAD = 10240
TROWS = NPAD // 16
RCH = 80
WCH = 64
ECH = 128
NCHUNK = E // ECH
NCH16 = NCHUNK // 16
NCHREM = NCHUNK % 16
LANES = 16

_MESH = dict(core_axis_name="c", subcore_axis_name="s")


def _qrsqrt(x):
    i = plsc.bitcast(x, jnp.int32)
    i = jnp.int32(0x5F3759DF) - (i >> 1)
    y = plsc.bitcast(i, jnp.float32)
    for _ in range(3):
        y = y * (1.5 - 0.5 * x * y * y)
    return y


def _zero_rows(buf, nrows):
    zero16 = jnp.zeros((LANES,), jnp.float32)

    @pl.loop(0, nrows)
    def _(r):
        for j in range(8):
            buf[r, pl.ds(j * 16, 16)] = zero16


def _scale_rows(wrows, sv, soff, ngroups, addrows=None):
    @pl.loop(0, ngroups)
    def _(gi):
        s16 = sv[pl.ds(soff + gi * 16, 16)]
        for r in range(16):
            row = gi * 16 + r
            s = s16[r]
            for j in range(8):
                sl = pl.ds(j * 16, 16)
                if addrows is None:
                    wrows[row, sl] = wrows[row, sl] * s
                else:
                    wrows[row, sl] = wrows[row, sl] * s + addrows[row, sl]


def _num_chunks(sid):
    return jnp.where(sid < NCHREM, NCH16 + 1, NCH16)


def _edge_pass(ei_hbm, g, sid, src_view, acc_sh, isrc, idst, rows):
    @pl.loop(0, _num_chunks(sid))
    def _(c):
        base = (c * 16 + sid) * ECH
        pltpu.sync_copy(ei_hbm.at[g, 0, 0, pl.ds(base, ECH)], isrc)
        pltpu.sync_copy(ei_hbm.at[g, 1, 0, pl.ds(base, ECH)], idst)
        pltpu.sync_copy(src_view.at[isrc], rows)
        pltpu.sync_copy(rows, acc_sh.at[idst], add=True)


def _sc_stats(x, ei):
    mesh = plsc.VectorSubcoreMesh(**_MESH)

    @functools.partial(
        pl.kernel,
        out_type=(
            jax.ShapeDtypeStruct((2, 4, 1, NPAD), jnp.float32),
            jax.ShapeDtypeStruct((2, NPAD, D), jnp.float32),
        ),
        mesh=mesh,
        compiler_params=pltpu.CompilerParams(needs_layout_passes=False),
        scratch_types=[
            pltpu.VMEM_SHARED((2, 16, 1, NPAD), jnp.float32),
            pltpu.VMEM((NPAD,), jnp.float32),
            pltpu.VMEM((NPAD,), jnp.float32),
            pltpu.VMEM((ECH,), jnp.int32),
            pltpu.VMEM((ECH,), jnp.int32),
            pltpu.VMEM((RCH, D), jnp.float32),
            pltpu.VMEM((TROWS,), jnp.float32),
            pltpu.VMEM((TROWS,), jnp.float32),
            pltpu.VMEM((TROWS,), jnp.float32),
            pltpu.VMEM((TROWS,), jnp.float32),
            pltpu.VMEM((TROWS,), jnp.float32),
            pltpu.VMEM((TROWS,), jnp.float32),
            pltpu.VMEM((TROWS,), jnp.float32),
        ],
    )
    def k(x_hbm, ei_hbm, scales_hbm, u0_hbm, stage_sh,
          hs, hd, isrc, idst, wrows, red, dout, din, av, bv, abv, invv):
        g = lax.axis_index("c")
        sid = lax.axis_index("s")
        rbase = sid * TROWS
        zero16 = jnp.zeros((LANES,), jnp.float32)
        ones16 = jnp.ones((LANES,), jnp.float32)

        @pl.loop(0, NPAD // 16)
        def _(j):
            hs[pl.ds(j * 16, 16)] = zero16
            hd[pl.ds(j * 16, 16)] = zero16

        @pl.loop(0, _num_chunks(sid))
        def _(c):
            base = (c * 16 + sid) * ECH
            pltpu.sync_copy(ei_hbm.at[g, 0, 0, pl.ds(base, ECH)], isrc)
            pltpu.sync_copy(ei_hbm.at[g, 1, 0, pl.ds(base, ECH)], idst)
            for j in range(8):
                plsc.addupdate_scatter(hs, [isrc[pl.ds(j * 16, 16)]], ones16)
                plsc.addupdate_scatter(hd, [idst[pl.ds(j * 16, 16)]], ones16)

        pltpu.sync_copy(hs, stage_sh.at[0, sid, 0])
        pltpu.sync_copy(hd, stage_sh.at[1, sid, 0])

        @pl.loop(0, TROWS // 16)
        def _(j):
            dout[pl.ds(j * 16, 16)] = zero16
            din[pl.ds(j * 16, 16)] = zero16

        plsc.subcore_barrier()

        @pl.loop(0, 16)
        def _(t):
            pltpu.sync_copy(stage_sh.at[0, t, 0, pl.ds(rbase, TROWS)], red)
            for j in range(TROWS // 16):
                sl = pl.ds(j * 16, 16)
                dout[sl] = dout[sl] + red[sl]
            pltpu.sync_copy(stage_sh.at[1, t, 0, pl.ds(rbase, TROWS)], red)
            for j in range(TROWS // 16):
                sl = pl.ds(j * 16, 16)
                din[sl] = din[sl] + red[sl]

        @pl.loop(0, TROWS // 16)
        def _(j):
            sl = pl.ds(j * 16, 16)
            dmo = jnp.maximum(dout[sl], 1.0)
            dmi = jnp.maximum(din[sl], 1.0)
            a = _qrsqrt(dmo)
            b = _qrsqrt(dmi)
            av[sl] = a
            bv[sl] = b
            abv[sl] = a * b
            invv[sl] = a * dmo

        pltpu.sync_copy(av, scales_hbm.at[g, 0, 0, pl.ds(rbase, TROWS)])
        pltpu.sync_copy(bv, scales_hbm.at[g, 1, 0, pl.ds(rbase, TROWS)])
        pltpu.sync_copy(abv, scales_hbm.at[g, 2, 0, pl.ds(rbase, TROWS)])
        pltpu.sync_copy(invv, scales_hbm.at[g, 3, 0, pl.ds(rbase, TROWS)])

        nch = jnp.where(sid == 15, (N - 15 * TROWS) // RCH, TROWS // RCH)

        @pl.loop(0, nch)
        def _(cc):
            r0 = rbase + cc * RCH
            pltpu.sync_copy(x_hbm.at[pl.ds(r0, RCH)], wrows)
            _scale_rows(wrows, av, cc * RCH, RCH // 16)
            pltpu.sync_copy(wrows, u0_hbm.at[g, pl.ds(r0, RCH)])

    return k(x, ei)


def _sc_front(ei, scales, u0):
    mesh = plsc.VectorSubcoreMesh(**_MESH)

    @functools.partial(
        pl.kernel,
        out_type=jax.ShapeDtypeStruct((2, K, NPAD, D), jnp.float32),
        mesh=mesh,
        compiler_params=pltpu.CompilerParams(needs_layout_passes=False),
        scratch_types=[
            pltpu.VMEM_SHARED((NPAD, D), jnp.float32),
            pltpu.VMEM((ECH,), jnp.int32),
            pltpu.VMEM((ECH,), jnp.int32),
            pltpu.VMEM((ECH, D), jnp.float32),
            pltpu.VMEM((WCH, D), jnp.float32),
            pltpu.VMEM((WCH, D), jnp.float32),
            pltpu.VMEM((TROWS,), jnp.float32),
        ],
    )
    def k(ei_hbm, scales_hbm, u0_hbm, v_hbm, acc_sh,
          isrc, idst, rows, wrows, zrows, abv):
        g = lax.axis_index("c")
        sid = lax.axis_index("s")
        rbase = sid * TROWS

        _zero_rows(zrows, WCH)
        pltpu.sync_copy(scales_hbm.at[g, 2, 0, pl.ds(rbase, TROWS)], abv)

        @pl.loop(0, TROWS // WCH)
        def _(cc):
            pltpu.sync_copy(zrows, acc_sh.at[pl.ds(rbase + cc * WCH, WCH)])

        plsc.subcore_barrier()

        for k_i in range(K):
            src_view = u0_hbm.at[g] if k_i == 0 else v_hbm.at[g, k_i - 1]
            _edge_pass(ei_hbm, g, sid, src_view, acc_sh, isrc, idst, rows)
            plsc.subcore_barrier()
            rezero = k_i < K - 1

            @pl.loop(0, TROWS // WCH)
            def _(cc):
                sl = pl.ds(rbase + cc * WCH, WCH)
                pltpu.sync_copy(acc_sh.at[sl], wrows)
                if rezero:
                    pltpu.sync_copy(zrows, acc_sh.at[sl])
                _scale_rows(wrows, abv, cc * WCH, WCH // 16)
                pltpu.sync_copy(wrows, v_hbm.at[g, k_i, sl])

            plsc.subcore_barrier()

    return k(ei, scales, u0)


def _sc_back(ei, scales, ya):
    mesh = plsc.VectorSubcoreMesh(**_MESH)

    @functools.partial(
        pl.kernel,
        out_type=(
            jax.ShapeDtypeStruct((2, NPAD, Z), jnp.float32),
            jax.ShapeDtypeStruct((2, NPAD, Z), jnp.float32),
        ),
        mesh=mesh,
        compiler_params=pltpu.CompilerParams(needs_layout_passes=False),
        scratch_types=[
            pltpu.VMEM_SHARED((NPAD, Z), jnp.float32),
            pltpu.VMEM((ECH,), jnp.int32),
            pltpu.VMEM((ECH,), jnp.int32),
            pltpu.VMEM((ECH, Z), jnp.float32),
            pltpu.VMEM((WCH, Z), jnp.float32),
            pltpu.VMEM((WCH, Z), jnp.float32),
            pltpu.VMEM((WCH, Z), jnp.float32),
            pltpu.VMEM((TROWS,), jnp.float32),
            pltpu.VMEM((TROWS,), jnp.float32),
        ],
    )
    def k(ei_hbm, scales_hbm, ya_hbm, tau_hbm, chain_hbm, acc_sh,
          isrc, idst, rows, wrows, arows, zrows, abv, bv):
        g = lax.axis_index("c")
        sid = lax.axis_index("s")
        rbase = sid * TROWS

        _zero_rows(zrows, WCH)
        pltpu.sync_copy(scales_hbm.at[g, 2, 0, pl.ds(rbase, TROWS)], abv)
        pltpu.sync_copy(scales_hbm.at[g, 1, 0, pl.ds(rbase, TROWS)], bv)

        @pl.loop(0, TROWS // WCH)
        def _(cc):
            pltpu.sync_copy(zrows, acc_sh.at[pl.ds(rbase + cc * WCH, WCH)])

        plsc.subcore_barrier()

        for k_i in range(K):
            src_view = ya_hbm.at[g, 2] if k_i == 0 else tau_hbm.at[g]
            _edge_pass(ei_hbm, g, sid, src_view, acc_sh, isrc, idst, rows)
            plsc.subcore_barrier()
            last = k_i == K - 1
            ya_j = 1 - k_i

            @pl.loop(0, TROWS // WCH)
            def _(cc):
                sl = pl.ds(rbase + cc * WCH, WCH)
                pltpu.sync_copy(acc_sh.at[sl], wrows)
                if not last:
                    pltpu.sync_copy(zrows, acc_sh.at[sl])
                    pltpu.sync_copy(ya_hbm.at[g, ya_j, sl], arows)
                    _scale_rows(wrows, abv, cc * WCH, WCH // 16, addrows=arows)
                    pltpu.sync_copy(wrows, tau_hbm.at[g, sl])
                else:
                    _scale_rows(wrows, bv, cc * WCH, WCH // 16)
                    pltpu.sync_copy(wrows, chain_hbm.at[g, sl])

            plsc.subcore_barrier()

    return k(ei, scales, ya)


ROWB = 400


def _mid_body(x_ref, v_ref, sc_ref, W1_ref, Wc_ref, s1_ref, c1_ref, b2_ref,
              ya_ref, zb_ref):
    g = pl.program_id(1)
    a = sc_ref[0, 0, :, 0]
    inva = sc_ref[0, 3, :, 0]
    x = x_ref[...]
    p1 = v_ref[0, 0] * inva[:, None]
    p2 = v_ref[0, 1] * inva[:, None]
    p3 = v_ref[0, 2] * inva[:, None]
    cat = jnp.concatenate([x, p1, p2, p3], axis=1)
    mm = jnp.dot(cat, W1_ref[0], preferred_element_type=jnp.float32)
    h = jnp.maximum(mm * s1_ref[0] + c1_ref[0], 0.0)
    big = jnp.dot(h, Wc_ref[0], preferred_element_type=jnp.float32)
    ya_ref[0, 0] = big[:, 0:Z] * a[:, None]
    ya_ref[0, 1] = big[:, Z:2 * Z] * a[:, None]
    ya_ref[0, 2] = big[:, 2 * Z:3 * Z] * a[:, None]
    zb = big[:, 3 * Z:4 * Z] + b2_ref[0]

    @pl.when(g == 0)
    def _():
        zb_ref[...] = zb

    @pl.when(g == 1)
    def _():
        zb_ref[...] = zb_ref[...] + zb


def _tc_mid(x, v, scales4, W1s, Wcs, s1s, c1s, b2s):
    grid = (N // ROWB, 2)
    return pl.pallas_call(
        _mid_body,
        grid=grid,
        in_specs=[
            pl.BlockSpec((ROWB, D), lambda i, g: (i, 0)),
            pl.BlockSpec((1, K, ROWB, D), lambda i, g: (g, 0, i, 0)),
            pl.BlockSpec((1, 4, ROWB, 1), lambda i, g: (g, 0, i, 0)),
            pl.BlockSpec((1, (K + 1) * D, H), lambda i, g: (g, 0, 0)),
            pl.BlockSpec((1, H, 4 * Z), lambda i, g: (g, 0, 0)),
            pl.BlockSpec((1, 1, H), lambda i, g: (g, 0, 0)),
            pl.BlockSpec((1, 1, H), lambda i, g: (g, 0, 0)),
            pl.BlockSpec((1, 1, Z), lambda i, g: (g, 0, 0)),
        ],
        out_specs=[
            pl.BlockSpec((1, K, ROWB, Z), lambda i, g: (g, 0, i, 0)),
            pl.BlockSpec((ROWB, Z), lambda i, g: (i, 0)),
        ],
        out_shape=[
            jax.ShapeDtypeStruct((2, K, NPAD, Z), jnp.float32),
            jax.ShapeDtypeStruct((N, Z), jnp.float32),
        ],
    )(x, v, scales4, W1s, Wcs, s1s, c1s, b2s)


def _sum_body(zb_ref, ch_ref, out_ref):
    out_ref[...] = zb_ref[...] + ch_ref[0] + ch_ref[1]


def _tc_sum(zb, chain):
    return pl.pallas_call(
        _sum_body,
        grid=(N // ROWB,),
        in_specs=[
            pl.BlockSpec((ROWB, Z), lambda i: (i, 0)),
            pl.BlockSpec((2, ROWB, Z), lambda i: (0, i, 0)),
        ],
        out_specs=pl.BlockSpec((ROWB, Z), lambda i: (i, 0)),
        out_shape=jax.ShapeDtypeStruct((N, Z), jnp.float32),
    )(zb, chain)


def kernel(x, G_edge_index, L_edge_index, W1_G, b1_G, W2_G, b2_G, Wm_G, bm_G,
           gamma_G, beta_G, mean_G, var_G, W1_L, b1_L, W2_L, b2_L, Wm_L, bm_L,
           gamma_L, beta_L, mean_L, var_L):
    ei = jnp.stack([G_edge_index, L_edge_index])[:, :, None, :]

    scales, u0 = _sc_stats(x, ei)
    v = _sc_front(ei, scales, u0)

    def bn_fold(gamma, beta, mean, var, b1):
        s = gamma * lax.rsqrt(var + 1e-3)
        return s, (b1 - mean) * s + beta

    s_G, c_G = bn_fold(gamma_G, beta_G, mean_G, var_G, b1_G)
    s_L, c_L = bn_fold(gamma_L, beta_L, mean_L, var_L, b1_L)
    W1s = jnp.stack([W1_G, W1_L])
    Wc_G = jnp.concatenate(
        [W2_G[H:2 * H], W2_G[2 * H:3 * H], W2_G[3 * H:], W2_G[:H] + Wm_G], axis=1)
    Wc_L = jnp.concatenate(
        [W2_L[H:2 * H], W2_L[2 * H:3 * H], W2_L[3 * H:], W2_L[:H] + Wm_L], axis=1)
    Wcs = jnp.stack([Wc_G, Wc_L])
    s1s = jnp.stack([s_G, s_L])[:, None, :]
    c1s = jnp.stack([c_G, c_L])[:, None, :]
    b2s = jnp.stack([b2_G + bm_G, b2_L + bm_L])[:, None, :]
    scales4 = scales.reshape(2, 4, NPAD, 1)

    ya, zb = _tc_mid(x, v, scales4, W1s, Wcs, s1s, c1s, b2s)
    _tau, chain = _sc_back(ei, scales, ya)
    return _tc_sum(zb, chain)


# speedup vs baseline: 7.7944x; 1.1489x over previous
import functools

import jax
import jax.numpy as jnp
from jax import lax
from jax.experimental import pallas as pl
from jax.experimental.pallas import tpu as pltpu
from jax.experimental.pallas import tpu_sc as plsc

N = 10000
E = 320000
D = 128
H = 256
Z = 128
K = 3

NPAD = 10240
TROWS = NPAD // 16
RCH = 80
WCH = 64
ECH = 128
UNI = 64
NCHUNK = E // ECH
NCH16 = NCHUNK // 16
NCHREM = NCHUNK % 16
LANES = 16

_MESH = dict(core_axis_name="c", subcore_axis_name="s")
_NOLAYOUT = dict(
    compiler_params=pltpu.CompilerParams(needs_layout_passes=False))


def _qrsqrt(x):
    i = plsc.bitcast(x, jnp.int32)
    i = jnp.int32(0x5F3759DF) - (i >> 1)
    y = plsc.bitcast(i, jnp.float32)
    for _ in range(3):
        y = y * (1.5 - 0.5 * x * y * y)
    return y


def _zero_rows(buf, nrows):
    zero16 = jnp.zeros((LANES,), jnp.float32)

    @pl.loop(0, nrows)
    def _(r):
        for j in range(8):
            buf[r, pl.ds(j * 16, 16)] = zero16


def _scale_rows(wrows, slot, sv, soff, ngroups, add_slot=None):
    @pl.loop(0, ngroups)
    def _(gi):
        s16 = sv[pl.ds(soff + gi * 16, 16)]
        for r in range(16):
            row = gi * 16 + r
            s = s16[r]
            for j in range(8):
                sl = pl.ds(j * 16, 16)
                if add_slot is None:
                    wrows[slot, row, sl] = wrows[slot, row, sl] * s
                else:
                    wrows[slot, row, sl] = (wrows[slot, row, sl] * s
                                            + wrows[add_slot, row, sl])


def _num_chunks(sid):
    return jnp.where(sid < NCHREM, NCH16 + 1, NCH16)


def _edge_pass(ei5, g, sid, src_view, acc_sh, isrc_s, idst_s, rows_s,
               gsem, ssems, isems):
    nuni = 2 * _num_chunks(sid)
    ngrp = (nuni + 3) // 4

    @pl.loop(0, ngrp)
    def _(t):
        for j in range(4):
            u = t * 4 + j
            p = j // 2

            @pl.when(u < nuni)
            def _():
                if j % 2 == 0:
                    @pl.when(u >= 4)
                    def _():
                        pltpu.make_async_copy(
                            rows_s.at[j], acc_sh.at[idst_s.at[j]],
                            ssems[j]).wait()
                        pltpu.make_async_copy(
                            rows_s.at[j + 1], acc_sh.at[idst_s.at[j + 1]],
                            ssems[j + 1]).wait()

                    cix = (t * 2 + p) * 16 + sid
                    d1 = pltpu.async_copy(
                        ei5.at[g, 0, cix], isrc_s.at[pl.ds(j, 2)], isems[p])
                    d2 = pltpu.async_copy(
                        ei5.at[g, 1, cix], idst_s.at[pl.ds(j, 2)], isems[p])
                    d1.wait()
                    d2.wait()

                pltpu.async_copy(
                    src_view.at[isrc_s.at[j]], rows_s.at[j], gsem).wait()
                pltpu.async_copy(
                    rows_s.at[j], acc_sh.at[idst_s.at[j]], ssems[j], add=True)

    for j in range(4):
        pltpu.make_async_copy(
            rows_s.at[j], acc_sh.at[idst_s.at[j]], ssems[j]).wait()


def _sc_stats(x, ei5):
    mesh = plsc.VectorSubcoreMesh(**_MESH)

    @functools.partial(
        pl.kernel,
        out_type=(
            jax.ShapeDtypeStruct((2, 4, 1, NPAD), jnp.float32),
            jax.ShapeDtypeStruct((2, NPAD, D), jnp.float32),
        ),
        mesh=mesh,
        scratch_types=[
            pltpu.VMEM_SHARED((2, 16, 1, NPAD), jnp.float32),
            pltpu.VMEM((NPAD,), jnp.float32),
            pltpu.VMEM((NPAD,), jnp.float32),
            pltpu.VMEM((2, 2, UNI), jnp.int32),
            pltpu.VMEM((2, 2, UNI), jnp.int32),
            pltpu.VMEM((RCH, D), jnp.float32),
            pltpu.VMEM((TROWS,), jnp.float32),
            pltpu.VMEM((TROWS,), jnp.float32),
            pltpu.VMEM((TROWS,), jnp.float32),
            pltpu.VMEM((TROWS,), jnp.float32),
            pltpu.VMEM((TROWS,), jnp.float32),
            pltpu.VMEM((TROWS,), jnp.float32),
            pltpu.VMEM((TROWS,), jnp.float32),
            pltpu.SemaphoreType.DMA,
            pltpu.SemaphoreType.DMA,
        ],
        **_NOLAYOUT,
    )
    def k(x_hbm, ei_hbm, scales_hbm, u0_hbm, stage_sh,
          hs, hd, isrc, idst, wrows, red, dout, din, av, bv, abv, invv,
          isem0, isem1):
        g = lax.axis_index("c")
        sid = lax.axis_index("s")
        rbase = sid * TROWS
        zero16 = jnp.zeros((LANES,), jnp.float32)
        ones16 = jnp.ones((LANES,), jnp.float32)
        isems = (isem0, isem1)
        nch = _num_chunks(sid)

        @pl.loop(0, NPAD // 16)
        def _(j):
            hs[pl.ds(j * 16, 16)] = zero16
            hd[pl.ds(j * 16, 16)] = zero16

        pltpu.async_copy(ei_hbm.at[g, 0, sid], isrc.at[0], isems[0])
        pltpu.async_copy(ei_hbm.at[g, 1, sid], idst.at[0], isems[0])

        @pl.loop(0, (nch + 1) // 2)
        def _(t):
            for b in range(2):
                c = t * 2 + b

                @pl.when(c < nch)
                def _():
                    pltpu.make_async_copy(
                        ei_hbm.at[g, 0, sid], isrc.at[b], isems[b]).wait()
                    pltpu.make_async_copy(
                        ei_hbm.at[g, 1, sid], idst.at[b], isems[b]).wait()

                    @pl.when(c + 1 < nch)
                    def _():
                        cix = (c + 1) * 16 + sid
                        pltpu.async_copy(
                            ei_hbm.at[g, 0, cix], isrc.at[1 - b], isems[1 - b])
                        pltpu.async_copy(
                            ei_hbm.at[g, 1, cix], idst.at[1 - b], isems[1 - b])

                    for rr in range(2):
                        for j in range(4):
                            sl = pl.ds(j * 16, 16)
                            plsc.addupdate_scatter(
                                hs, [isrc[b, rr, sl]], ones16)
                            plsc.addupdate_scatter(
                                hd, [idst[b, rr, sl]], ones16)

        pltpu.sync_copy(hs, stage_sh.at[0, sid, 0])
        pltpu.sync_copy(hd, stage_sh.at[1, sid, 0])

        @pl.loop(0, TROWS // 16)
        def _(j):
            dout[pl.ds(j * 16, 16)] = zero16
            din[pl.ds(j * 16, 16)] = zero16

        plsc.subcore_barrier()

        @pl.loop(0, 16)
        def _(t):
            pltpu.sync_copy(stage_sh.at[0, t, 0, pl.ds(rbase, TROWS)], red)
            for j in range(TROWS // 16):
                sl = pl.ds(j * 16, 16)
                dout[sl] = dout[sl] + red[sl]
            pltpu.sync_copy(stage_sh.at[1, t, 0, pl.ds(rbase, TROWS)], red)
            for j in range(TROWS // 16):
                sl = pl.ds(j * 16, 16)
                din[sl] = din[sl] + red[sl]

        @pl.loop(0, TROWS // 16)
        def _(j):
            sl = pl.ds(j * 16, 16)
            dmo = jnp.maximum(dout[sl], 1.0)
            dmi = jnp.maximum(din[sl], 1.0)
            a = _qrsqrt(dmo)
            b = _qrsqrt(dmi)
            av[sl] = a
            bv[sl] = b
            abv[sl] = a * b
            invv[sl] = a * dmo

        pltpu.sync_copy(av, scales_hbm.at[g, 0, 0, pl.ds(rbase, TROWS)])
        pltpu.sync_copy(bv, scales_hbm.at[g, 1, 0, pl.ds(rbase, TROWS)])
        pltpu.sync_copy(abv, scales_hbm.at[g, 2, 0, pl.ds(rbase, TROWS)])
        pltpu.sync_copy(invv, scales_hbm.at[g, 3, 0, pl.ds(rbase, TROWS)])

        nrch = jnp.where(sid == 15, (N - 15 * TROWS) // RCH, TROWS // RCH)

        @pl.loop(0, nrch)
        def _(cc):
            r0 = rbase + cc * RCH
            pltpu.sync_copy(x_hbm.at[pl.ds(r0, RCH)], wrows)

            @pl.loop(0, RCH // 16)
            def _(gi):
                s16 = av[pl.ds(cc * RCH + gi * 16, 16)]
                for r in range(16):
                    row = gi * 16 + r
                    s = s16[r]
                    for j in range(8):
                        sl = pl.ds(j * 16, 16)
                        wrows[row, sl] = wrows[row, sl] * s

            pltpu.sync_copy(wrows, u0_hbm.at[g, pl.ds(r0, RCH)])

    return k(x, ei5)


def _prop_scratch(width):
    return [
        pltpu.VMEM_SHARED((NPAD, width), jnp.float32),
        pltpu.VMEM((4, UNI), jnp.int32),
        pltpu.VMEM((4, UNI), jnp.int32),
        pltpu.VMEM((4, UNI, width), jnp.float32),
        pltpu.VMEM((WCH, width), jnp.float32),
        pltpu.VMEM((TROWS,), jnp.float32),
        pltpu.SemaphoreType.DMA,
        pltpu.SemaphoreType.DMA,
        pltpu.SemaphoreType.DMA,
        pltpu.SemaphoreType.DMA,
        pltpu.SemaphoreType.DMA,
        pltpu.SemaphoreType.DMA,
        pltpu.SemaphoreType.DMA,
    ]


def _sc_front(ei5, scales, u0):
    mesh = plsc.VectorSubcoreMesh(**_MESH)

    @functools.partial(
        pl.kernel,
        out_type=jax.ShapeDtypeStruct((2, K, NPAD, D), jnp.float32),
        mesh=mesh,
        scratch_types=_prop_scratch(D),
        **_NOLAYOUT,
    )
    def k(ei_hbm, scales_hbm, u0_hbm, v_hbm, acc_sh,
          isrc_s, idst_s, rows_s, zrows, abv,
          gsem, ss0, ss1, ss2, ss3, is0, is1):
        g = lax.axis_index("c")
        sid = lax.axis_index("s")
        rbase = sid * TROWS
        ssems = (ss0, ss1, ss2, ss3)
        isems = (is0, is1)

        _zero_rows(zrows, WCH)
        pltpu.sync_copy(scales_hbm.at[g, 2, 0, pl.ds(rbase, TROWS)], abv)

        @pl.loop(0, TROWS // WCH)
        def _(cc):
            pltpu.sync_copy(zrows, acc_sh.at[pl.ds(rbase + cc * WCH, WCH)])

        plsc.subcore_barrier()

        for k_i in range(K):
            src_view = u0_hbm.at[g] if k_i == 0 else v_hbm.at[g, k_i - 1]
            _edge_pass(ei_hbm, g, sid, src_view, acc_sh, isrc_s, idst_s,
                       rows_s, gsem, ssems, isems)
            plsc.subcore_barrier()
            rezero = k_i < K - 1

            @pl.loop(0, TROWS // WCH)
            def _(cc):
                sl = pl.ds(rbase + cc * WCH, WCH)
                pltpu.sync_copy(acc_sh.at[sl], rows_s.at[0])
                if rezero:
                    pltpu.sync_copy(zrows, acc_sh.at[sl])
                _scale_rows(rows_s, 0, abv, cc * WCH, WCH // 16)
                pltpu.sync_copy(rows_s.at[0], v_hbm.at[g, k_i, sl])

            plsc.subcore_barrier()

    return k(ei5, scales, u0)


def _sc_back(ei5, scales, ya):
    mesh = plsc.VectorSubcoreMesh(**_MESH)

    @functools.partial(
        pl.kernel,
        out_type=(
            jax.ShapeDtypeStruct((2, NPAD, Z), jnp.float32),
            jax.ShapeDtypeStruct((2, NPAD, Z), jnp.float32),
        ),
        mesh=mesh,
        scratch_types=_prop_scratch(Z) + [
            pltpu.VMEM((TROWS,), jnp.float32),
        ],
        **_NOLAYOUT,
    )
    def k(ei_hbm, scales_hbm, ya_hbm, tau_hbm, chain_hbm, acc_sh,
          isrc_s, idst_s, rows_s, zrows, abv,
          gsem, ss0, ss1, ss2, ss3, is0, is1, bv):
        g = lax.axis_index("c")
        sid = lax.axis_index("s")
        rbase = sid * TROWS
        ssems = (ss0, ss1, ss2, ss3)
        isems = (is0, is1)

        _zero_rows(zrows, WCH)
        pltpu.sync_copy(scales_hbm.at[g, 2, 0, pl.ds(rbase, TROWS)], abv)
        pltpu.sync_copy(scales_hbm.at[g, 1, 0, pl.ds(rbase, TROWS)], bv)

        @pl.loop(0, TROWS // WCH)
        def _(cc):
            pltpu.sync_copy(zrows, acc_sh.at[pl.ds(rbase + cc * WCH, WCH)])

        plsc.subcore_barrier()

        for k_i in range(K):
            src_view = ya_hbm.at[g, 2] if k_i == 0 else tau_hbm.at[g]
            _edge_pass(ei_hbm, g, sid, src_view, acc_sh, isrc_s, idst_s,
                       rows_s, gsem, ssems, isems)
            plsc.subcore_barrier()
            last = k_i == K - 1
            ya_j = 1 - k_i

            @pl.loop(0, TROWS // WCH)
            def _(cc):
                sl = pl.ds(rbase + cc * WCH, WCH)
                pltpu.sync_copy(acc_sh.at[sl], rows_s.at[0])
                if not last:
                    pltpu.sync_copy(zrows, acc_sh.at[sl])
                    pltpu.sync_copy(ya_hbm.at[g, ya_j, sl], rows_s.at[1])
                    _scale_rows(rows_s, 0, abv, cc * WCH, WCH // 16,
                                add_slot=1)
                    pltpu.sync_copy(rows_s.at[0], tau_hbm.at[g, sl])
                else:
                    _scale_rows(rows_s, 0, bv, cc * WCH, WCH // 16)
                    pltpu.sync_copy(rows_s.at[0], chain_hbm.at[g, sl])

            plsc.subcore_barrier()

    return k(ei5, scales, ya)


ROWB = 400


def _mid_body(x_ref, v_ref, sc_ref, W1_ref, Wc_ref, s1_ref, c1_ref, b2_ref,
              ya_ref, zb_ref):
    g = pl.program_id(1)
    a = sc_ref[0, 0, :, 0]
    inva = sc_ref[0, 3, :, 0]
    x = x_ref[...]
    p1 = v_ref[0, 0] * inva[:, None]
    p2 = v_ref[0, 1] * inva[:, None]
    p3 = v_ref[0, 2] * inva[:, None]
    cat = jnp.concatenate([x, p1, p2, p3], axis=1)
    mm = jnp.dot(cat, W1_ref[0], preferred_element_type=jnp.float32)
    h = jnp.maximum(mm * s1_ref[0] + c1_ref[0], 0.0)
    big = jnp.dot(h, Wc_ref[0], preferred_element_type=jnp.float32)
    ya_ref[0, 0] = big[:, 0:Z] * a[:, None]
    ya_ref[0, 1] = big[:, Z:2 * Z] * a[:, None]
    ya_ref[0, 2] = big[:, 2 * Z:3 * Z] * a[:, None]
    zb = big[:, 3 * Z:4 * Z] + b2_ref[0]

    @pl.when(g == 0)
    def _():
        zb_ref[...] = zb

    @pl.when(g == 1)
    def _():
        zb_ref[...] = zb_ref[...] + zb


def _tc_mid(x, v, scales4, W1s, Wcs, s1s, c1s, b2s):
    grid = (N // ROWB, 2)
    return pl.pallas_call(
        _mid_body,
        grid=grid,
        in_specs=[
            pl.BlockSpec((ROWB, D), lambda i, g: (i, 0)),
            pl.BlockSpec((1, K, ROWB, D), lambda i, g: (g, 0, i, 0)),
            pl.BlockSpec((1, 4, ROWB, 1), lambda i, g: (g, 0, i, 0)),
            pl.BlockSpec((1, (K + 1) * D, H), lambda i, g: (g, 0, 0)),
            pl.BlockSpec((1, H, 4 * Z), lambda i, g: (g, 0, 0)),
            pl.BlockSpec((1, 1, H), lambda i, g: (g, 0, 0)),
            pl.BlockSpec((1, 1, H), lambda i, g: (g, 0, 0)),
            pl.BlockSpec((1, 1, Z), lambda i, g: (g, 0, 0)),
        ],
        out_specs=[
            pl.BlockSpec((1, K, ROWB, Z), lambda i, g: (g, 0, i, 0)),
            pl.BlockSpec((ROWB, Z), lambda i, g: (i, 0)),
        ],
        out_shape=[
            jax.ShapeDtypeStruct((2, K, NPAD, Z), jnp.float32),
            jax.ShapeDtypeStruct((N, Z), jnp.float32),
        ],
    )(x, v, scales4, W1s, Wcs, s1s, c1s, b2s)


def _sum_body(zb_ref, ch_ref, out_ref):
    out_ref[...] = zb_ref[...] + ch_ref[0] + ch_ref[1]


def _tc_sum(zb, chain):
    return pl.pallas_call(
        _sum_body,
        grid=(N // ROWB,),
        in_specs=[
            pl.BlockSpec((ROWB, Z), lambda i: (i, 0)),
            pl.BlockSpec((2, ROWB, Z), lambda i: (0, i, 0)),
        ],
        out_specs=pl.BlockSpec((ROWB, Z), lambda i: (i, 0)),
        out_shape=jax.ShapeDtypeStruct((N, Z), jnp.float32),
    )(zb, chain)


def kernel(x, G_edge_index, L_edge_index, W1_G, b1_G, W2_G, b2_G, Wm_G, bm_G,
           gamma_G, beta_G, mean_G, var_G, W1_L, b1_L, W2_L, b2_L, Wm_L, bm_L,
           gamma_L, beta_L, mean_L, var_L):
    ei5 = jnp.stack([G_edge_index, L_edge_index]).reshape(
        2, 2, NCHUNK, 2, UNI)

    scales, u0 = _sc_stats(x, ei5)
    v = _sc_front(ei5, scales, u0)

    def bn_fold(gamma, beta, mean, var, b1):
        s = gamma * lax.rsqrt(var + 1e-3)
        return s, (b1 - mean) * s + beta

    s_G, c_G = bn_fold(gamma_G, beta_G, mean_G, var_G, b1_G)
    s_L, c_L = bn_fold(gamma_L, beta_L, mean_L, var_L, b1_L)
    W1s = jnp.stack([W1_G, W1_L])
    Wc_G = jnp.concatenate(
        [W2_G[H:2 * H], W2_G[2 * H:3 * H], W2_G[3 * H:], W2_G[:H] + Wm_G], axis=1)
    Wc_L = jnp.concatenate(
        [W2_L[H:2 * H], W2_L[2 * H:3 * H], W2_L[3 * H:], W2_L[:H] + Wm_L], axis=1)
    Wcs = jnp.stack([Wc_G, Wc_L])
    s1s = jnp.stack([s_G, s_L])[:, None, :]
    c1s = jnp.stack([c_G, c_L])[:, None, :]
    b2s = jnp.stack([b2_G + bm_G, b2_L + bm_L])[:, None, :]
    scales4 = scales.reshape(2, 4, NPAD, 1)

    ya, zb = _tc_mid(x, v, scales4, W1s, Wcs, s1s, c1s, b2s)
    _tau, chain = _sc_back(ei5, scales, ya)
    return _tc_sum(zb, chain)
